# baseline (speedup 1.0000x reference)
import jax
import jax.numpy as jnp
from jax import lax
from jax.experimental import pallas as pl
from jax.experimental.pallas import tpu as pltpu
from jax.experimental.pallas import tpu_sc as plsc

BATCH = 16384
DIM = 32
NBOUND = 1000
NBPAD = 1024
NC = 2
NS = 16
NW = NC * NS
BPW = BATCH // NW
NCHUNK = 4
CHUNK = BPW // NCHUNK
L = 16
OUTW = 128
PACK = OUTW // DIM

ULINES = (VOCAB_LINES := (1000000 * DIM + 31) // OUTW + 1)
TLINES = ((NBOUND + 1) * DIM + OUTW - 1) // OUTW


def _pack_lines(table, nlines):
    flat = table.reshape(-1)
    pad = nlines * OUTW - flat.shape[0]
    return jnp.pad(flat, (0, pad)).reshape(nlines, OUTW)


def _body(uid_hbm, ts_hbm, utab_hbm, ttab_hbm, bnd_hbm, mean_hbm, istd_hbm,
          out_hbm,
          idx_v, bkt_v, ublk_v, tblk_v, ts_v, bnds_v, mean_v, istd_v,
          out_v, gbuf0, gbuf1, sem0, sem1):
    c = lax.axis_index("c")
    s = lax.axis_index("s")
    wid = s * NC + c
    base = wid * BPW
    gbufs = (gbuf0, gbuf1)
    sems = (sem0, sem1)

    pltpu.sync_copy(uid_hbm.at[wid], idx_v)
    for p in range(BPW // L):
        v = idx_v[0, pl.ds(p * L, L)]
        ublk_v[p // 8, pl.ds((p % 8) * L, L)] = v >> 2

    def fire(i):
        if i < NCHUNK:
            return pltpu.async_copy(utab_hbm.at[ublk_v.at[i]],
                                    gbufs[i % 2], sems[i % 2])
        return pltpu.async_copy(ttab_hbm.at[tblk_v.at[i - NCHUNK]],
                                gbufs[i % 2], sems[i % 2])

    descs = {0: fire(0), 1: fire(1)}

    pltpu.sync_copy(ts_hbm.at[wid], ts_v)
    pltpu.sync_copy(bnd_hbm, bnds_v)
    pltpu.sync_copy(mean_hbm, mean_v)
    pltpu.sync_copy(istd_hbm, istd_v)
    mean = mean_v[...]
    istd = istd_v[...]

    norm_col = jnp.full((L,), 2 * DIM, jnp.int32)
    for p in range(BPW // L):
        tf = ts_v[0, pl.ds(p * L, L)].astype(jnp.float32)
        pos = jnp.zeros((L,), jnp.int32)
        for w in (512, 256, 128, 64, 32, 16, 8, 4, 2, 1):
            probe = plsc.load_gather(bnds_v, [pos + (w - 1)])
            pos = jnp.where(probe <= tf, pos + w, pos)
        bkt_v[0, pl.ds(p * L, L)] = pos
        tblk_v[p // 8, pl.ds((p % 8) * L, L)] = pos >> 2
        rows = p * L + jnp.arange(L, dtype=jnp.int32)
        plsc.store_scatter(out_v, [rows, norm_col], (tf - mean) * istd)

    lanes = jnp.arange(L, dtype=jnp.int32)

    for i in range(2 * NCHUNK):
        descs.pop(i).wait()
        buf = gbufs[i % 2]
        user_side = i < NCHUNK
        row_base = (i % NCHUNK) * CHUNK
        col0 = 0 if user_side else DIM
        src_idx = idx_v if user_side else bkt_v

        def extract(p, carry, buf=buf, row_base=row_base, col0=col0,
                    src_idx=src_idx):
            rloc = p * L + lanes
            rows = row_base + rloc
            ids = src_idx[0, pl.ds(row_base + p * L, L)]
            sub = (ids & (PACK - 1)) << 5
            for col in range(DIM):
                vals = plsc.load_gather(buf, [rloc, sub + col])
                plsc.store_scatter(
                    out_v, [rows, jnp.full((L,), col0 + col, jnp.int32)],
                    vals)
            return carry
        lax.fori_loop(0, CHUNK // L, extract, 0)
        if i + 2 < 2 * NCHUNK:
            descs[i + 2] = fire(i + 2)

    pltpu.sync_copy(out_v, out_hbm.at[pl.ds(base, BPW)])


def kernel(userID, review_date_in_unix, user_table, ts_table, boundaries,
           ts_mean, ts_std):
    uid = userID.reshape(NW, 1, BPW)
    ts = review_date_in_unix.reshape(NW, 1, BPW)
    utab = _pack_lines(user_table, ULINES)
    ttab = _pack_lines(ts_table, TLINES)
    bpad = jnp.concatenate([
        boundaries.astype(jnp.float32),
        jnp.full((NBPAD - NBOUND,), jnp.inf, jnp.float32),
    ])
    mean_v = jnp.broadcast_to(ts_mean.astype(jnp.float32), (L,))
    istd_v = jnp.broadcast_to((1.0 / ts_std).astype(jnp.float32), (L,))

    mesh = plsc.VectorSubcoreMesh(core_axis_name="c", subcore_axis_name="s")
    run = pl.kernel(
        _body,
        out_type=jax.ShapeDtypeStruct((BATCH, OUTW), jnp.float32),
        mesh=mesh,
        scratch_types=[
            pltpu.VMEM((1, BPW), jnp.int32),
            pltpu.VMEM((1, BPW), jnp.int32),
            pltpu.VMEM((NCHUNK, CHUNK), jnp.int32),
            pltpu.VMEM((NCHUNK, CHUNK), jnp.int32),
            pltpu.VMEM((1, BPW), jnp.int32),
            pltpu.VMEM((NBPAD,), jnp.float32),
            pltpu.VMEM((L,), jnp.float32),
            pltpu.VMEM((L,), jnp.float32),
            pltpu.VMEM((BPW, OUTW), jnp.float32),
            pltpu.VMEM((CHUNK, OUTW), jnp.float32),
            pltpu.VMEM((CHUNK, OUTW), jnp.float32),
            pltpu.SemaphoreType.DMA,
            pltpu.SemaphoreType.DMA,
        ],
        compiler_params=pltpu.CompilerParams(needs_layout_passes=False),
    )
    out = run(uid, ts, utab, ttab, bpad, mean_v, istd_v)
    return out[:, : 2 * DIM + 1]


# speedup vs baseline: 1.2131x; 1.2131x over previous
import jax
import jax.numpy as jnp
from jax import lax
from jax.experimental import pallas as pl
from jax.experimental.pallas import tpu as pltpu
from jax.experimental.pallas import tpu_sc as plsc

BATCH = 16384
DIM = 32
NBOUND = 1000
NBPAD = 1024
NC = 2
NS = 16
NW = NC * NS
BPW = BATCH // NW
L = 16
OUTW = 128


def _body(uid_hbm, ts_hbm, utab_hbm, ttab_hbm, bnd_hbm, mean_hbm, istd_hbm,
          out_hbm,
          idx_v, ts_v, bnds_v, mean_v, istd_v, out_v, usem, tsem):
    c = lax.axis_index("c")
    s = lax.axis_index("s")
    wid = s * NC + c
    base = wid * BPW

    def fire_row(tab_hbm, row_id, r, col0, sem):
        off = pl.multiple_of(row_id * DIM, 8)
        pltpu.async_copy(tab_hbm.at[pl.ds(off, DIM)],
                         out_v.at[r, pl.ds(col0, DIM)], sem)

    pltpu.sync_copy(uid_hbm.at[wid], idx_v)

    def ufire(g, carry):
        vec = idx_v[0, pl.ds(g * L, L)]
        for b in range(L):
            fire_row(utab_hbm, vec[b], g * L + b, 0, usem)
        return carry
    lax.fori_loop(0, BPW // L, ufire, 0)

    pltpu.sync_copy(ts_hbm.at[wid], ts_v)
    pltpu.sync_copy(bnd_hbm, bnds_v)
    pltpu.sync_copy(mean_hbm, mean_v)
    pltpu.sync_copy(istd_hbm, istd_v)
    mean = mean_v[...]
    istd = istd_v[...]

    norm_col = jnp.full((L,), 2 * DIM, jnp.int32)
    for p in range(BPW // L):
        tf = ts_v[0, pl.ds(p * L, L)].astype(jnp.float32)
        pos = jnp.zeros((L,), jnp.int32)
        for w in (512, 256, 128, 64, 32, 16, 8, 4, 2, 1):
            probe = plsc.load_gather(bnds_v, [pos + (w - 1)])
            pos = jnp.where(probe <= tf, pos + w, pos)
        for b in range(L):
            fire_row(ttab_hbm, pos[b], p * L + b, DIM, tsem)
        rows = p * L + jnp.arange(L, dtype=jnp.int32)
        plsc.store_scatter(out_v, [rows, norm_col], (tf - mean) * istd)

    pltpu.make_async_copy(out_hbm.at[pl.ds(0, BPW // 4)],
                          out_v.at[pl.ds(0, BPW // 4), :], usem).wait()
    pltpu.make_async_copy(out_hbm.at[pl.ds(0, BPW // 4)],
                          out_v.at[pl.ds(0, BPW // 4), :], tsem).wait()

    pltpu.sync_copy(out_v, out_hbm.at[pl.ds(base, BPW)])


def kernel(userID, review_date_in_unix, user_table, ts_table, boundaries,
           ts_mean, ts_std):
    uid = userID.reshape(NW, 1, BPW)
    ts = review_date_in_unix.reshape(NW, 1, BPW)
    utab = user_table.reshape(-1)
    ttab = ts_table.reshape(-1)
    bpad = jnp.concatenate([
        boundaries.astype(jnp.float32),
        jnp.full((NBPAD - NBOUND,), jnp.inf, jnp.float32),
    ])
    mean_v = jnp.broadcast_to(ts_mean.astype(jnp.float32), (L,))
    istd_v = jnp.broadcast_to((1.0 / ts_std).astype(jnp.float32), (L,))

    mesh = plsc.VectorSubcoreMesh(core_axis_name="c", subcore_axis_name="s")
    run = pl.kernel(
        _body,
        out_type=jax.ShapeDtypeStruct((BATCH, OUTW), jnp.float32),
        mesh=mesh,
        scratch_types=[
            pltpu.VMEM((1, BPW), jnp.int32),
            pltpu.VMEM((1, BPW), jnp.int32),
            pltpu.VMEM((NBPAD,), jnp.float32),
            pltpu.VMEM((L,), jnp.float32),
            pltpu.VMEM((L,), jnp.float32),
            pltpu.VMEM((BPW, OUTW), jnp.float32),
            pltpu.SemaphoreType.DMA,
            pltpu.SemaphoreType.DMA,
        ],
        compiler_params=pltpu.CompilerParams(needs_layout_passes=False),
    )
    out = run(uid, ts, utab, ttab, bpad, mean_v, istd_v)
    return out[:, : 2 * DIM + 1]
